# Initial kernel scaffold; baseline (speedup 1.0000x reference)
import jax
import jax.numpy as jnp
from jax.experimental import pallas as pl


def kernel(input, target):
    raise NotImplementedError("write your pallas kernel here")


# speedup vs baseline: 1.4531x; 1.4531x over previous
import functools
import math

import jax
import jax.numpy as jnp
from jax import lax
from jax.experimental import pallas as pl
from jax.experimental.pallas import tpu as pltpu
from jax.experimental.pallas import tpu_sc as plsc

_TOPK = 5
_LN_POWER = math.log(1.01)
_L = 16
_B = 128
_V = 100000
_NVEC = _V // _L
_NW = 32
_ROWS_PER_W = _B // _NW
_LN2 = 0.6931471805599453


def _take(x, idx):
    return x.at[idx].get(mode="promise_in_bounds")


def _xlane_sum(x):
    iota = lax.iota(jnp.int32, _L)
    for k in (8, 4, 2, 1):
        x = x + _take(x, iota ^ k)
    return x


def _xlane_max(x):
    iota = lax.iota(jnp.int32, _L)
    for k in (8, 4, 2, 1):
        x = jnp.maximum(x, _take(x, iota ^ k))
    return x


def _vlog(x):
    bits = plsc.bitcast(x, jnp.int32)
    e = lax.shift_right_logical(bits, 23) - 127
    mbits = (bits & 0x7FFFFF) | (127 << 23)
    m = plsc.bitcast(mbits, jnp.float32)
    big = m > 1.4142135623730951
    m = jnp.where(big, m * 0.5, m)
    e = e + big.astype(jnp.int32)
    t = (m - 1.0) / (m + 1.0)
    t2 = t * t
    p = 2.0 * t * (1.0 + t2 * (1.0 / 3.0 + t2 * (0.2 + t2 * (1.0 / 7.0))))
    return e.astype(jnp.float32) * _LN2 + p


def _sc_loss_call(inp, target):
    mesh = plsc.VectorSubcoreMesh(core_axis_name="c", subcore_axis_name="s")

    @functools.partial(
        pl.kernel,
        out_type=jax.ShapeDtypeStruct((_NW, _L), jnp.float32),
        mesh=mesh,
        compiler_params=pltpu.CompilerParams(needs_layout_passes=False),
        scratch_types=[
            pltpu.VMEM((_V,), jnp.float32),
            pltpu.VMEM((_B,), jnp.int32),
            pltpu.VMEM((_L,), jnp.float32),
        ],
    )
    def run(inp_hbm, tgt_hbm, out_hbm, row_v, tgt_v, out_v):
        wid = lax.axis_index("s") * 2 + lax.axis_index("c")
        iota = lax.iota(jnp.int32, _L)
        neg_inf = jnp.full((_L,), -jnp.inf, jnp.float32)

        pltpu.sync_copy(tgt_hbm, tgt_v)
        w0 = (wid // 4) * 16
        tgt_vec = tgt_v[pl.ds(w0, _L)]
        lane_base = (wid % 4) * 4

        out_acc = jnp.zeros((_L,), jnp.float32)
        for j in range(_ROWS_PER_W):
            r = wid * _ROWS_PER_W + j
            pltpu.sync_copy(inp_hbm.at[r], row_v)

            lane = lane_base + j
            t_f = _xlane_sum(jnp.where(iota == lane, tgt_vec, 0)
                             .astype(jnp.float32))
            t_b = t_f.astype(jnp.int32)
            x_t = plsc.load_gather(row_v, [t_b])

            def pass_a(i, carry):
                t1, t2_, t3, t4, t5, cnt = carry
                v = row_v[pl.ds(i * _L, _L)]
                idx = iota + i * _L
                pred = (v > x_t) | ((v == x_t) & (idx < t_b))
                cnt = cnt + pred.astype(jnp.float32)
                m1 = jnp.maximum(t1, v)
                r1 = jnp.minimum(t1, v)
                m2 = jnp.maximum(t2_, r1)
                r2 = jnp.minimum(t2_, r1)
                m3 = jnp.maximum(t3, r2)
                r3 = jnp.minimum(t3, r2)
                m4 = jnp.maximum(t4, r3)
                r4 = jnp.minimum(t4, r3)
                m5 = jnp.maximum(t5, r4)
                return (m1, m2, m3, m4, m5, cnt)

            t1, t2_, t3, t4, t5, cnt = lax.fori_loop(
                0, _NVEC, pass_a,
                (neg_inf, neg_inf, neg_inf, neg_inf, neg_inf,
                 jnp.zeros((_L,), jnp.float32)),
                unroll=2)
            rank = _xlane_sum(cnt)

            m_b = _xlane_max(t1)

            def pass_b(i, s):
                v = row_v[pl.ds(i * _L, _L)]
                return s + jnp.exp(v - m_b)

            s16 = lax.fori_loop(0, _NVEC, pass_b,
                                jnp.zeros((_L,), jnp.float32), unroll=2)
            s_b = _xlane_sum(s16)

            vv = neg_inf
            for k in range(_TOPK):
                topv_b = _xlane_max(t1)
                vv = jnp.where(iota == k, topv_b, vv)
                if k < _TOPK - 1:
                    hit = t1 == topv_b
                    first = iota == plsc.all_reduce_ffs(hit)
                    t1 = jnp.where(first, t2_, t1)
                    t2_ = jnp.where(first, t3, t2_)
                    t3 = jnp.where(first, t4, t3)
                    t4 = jnp.where(first, t5, t4)
                    t5 = jnp.where(first, neg_inf, t5)

            pw = jnp.exp(vv * _LN_POWER)
            s5_b = _xlane_sum(pw)

            lse = m_b + _vlog(s_b)
            in_topk = rank < float(_TOPK)
            val2 = jnp.where(in_topk, x_t * _LN_POWER - _vlog(s5_b),
                             x_t - lse)
            contrib = -(x_t - lse) - val2
            out_acc = jnp.where(iota == j, contrib, out_acc)

        out_v[...] = out_acc
        pltpu.sync_copy(out_v, out_hbm.at[wid])

    return run(inp, target)


def kernel(input, target):
    per_worker = _sc_loss_call(input, target)
    per_row = per_worker[:, :_ROWS_PER_W].reshape(_B)
    return jnp.mean(per_row)


# speedup vs baseline: 2.0762x; 1.4287x over previous
import functools
import math

import jax
import jax.numpy as jnp
from jax import lax
from jax.experimental import pallas as pl
from jax.experimental.pallas import tpu as pltpu
from jax.experimental.pallas import tpu_sc as plsc

_TOPK = 5
_LN_POWER = math.log(1.01)
_L = 16
_B = 128
_V = 100000
_NVEC = _V // _L
_NW = 32
_ROWS_PER_W = _B // _NW
_LN2 = 0.6931471805599453
_U = 10
_NCHUNK = 16
_CSIZE = 390


def _take(x, idx):
    return x.at[idx].get(mode="promise_in_bounds")


def _xlane_sum(x):
    iota = lax.iota(jnp.int32, _L)
    for k in (8, 4, 2, 1):
        x = x + _take(x, iota ^ k)
    return x


def _xlane_max(x):
    iota = lax.iota(jnp.int32, _L)
    for k in (8, 4, 2, 1):
        x = jnp.maximum(x, _take(x, iota ^ k))
    return x


def _vlog(x):
    bits = plsc.bitcast(x, jnp.int32)
    e = lax.shift_right_logical(bits, 23) - 127
    mbits = (bits & 0x7FFFFF) | (127 << 23)
    m = plsc.bitcast(mbits, jnp.float32)
    big = m > 1.4142135623730951
    m = jnp.where(big, m * 0.5, m)
    e = e + big.astype(jnp.int32)
    t = (m - 1.0) / (m + 1.0)
    t2 = t * t
    p = 2.0 * t * (1.0 + t2 * (1.0 / 3.0 + t2 * (0.2 + t2 * (1.0 / 7.0))))
    return e.astype(jnp.float32) * _LN2 + p


def _tree_max(vs):
    while len(vs) > 1:
        vs = [jnp.maximum(vs[i], vs[i + 1]) for i in range(0, len(vs) - 1, 2)] \
            + ([vs[-1]] if len(vs) % 2 else [])
    return vs[0]


def _tree_sum(vs):
    while len(vs) > 1:
        vs = [vs[i] + vs[i + 1] for i in range(0, len(vs) - 1, 2)] \
            + ([vs[-1]] if len(vs) % 2 else [])
    return vs[0]


def _sc_loss_call(inp, target):
    mesh = plsc.VectorSubcoreMesh(core_axis_name="c", subcore_axis_name="s")

    @functools.partial(
        pl.kernel,
        out_type=jax.ShapeDtypeStruct((_NW, _L), jnp.float32),
        mesh=mesh,
        compiler_params=pltpu.CompilerParams(needs_layout_passes=False),
        scratch_types=[
            pltpu.VMEM((_V,), jnp.float32),
            pltpu.VMEM((_B,), jnp.int32),
            pltpu.VMEM((_L,), jnp.float32),
        ],
    )
    def run(inp_hbm, tgt_hbm, out_hbm, row_v, tgt_v, out_v):
        wid = lax.axis_index("s") * 2 + lax.axis_index("c")
        iota = lax.iota(jnp.int32, _L)
        neg_inf = jnp.full((_L,), -jnp.inf, jnp.float32)

        pltpu.sync_copy(tgt_hbm, tgt_v)
        w0 = (wid // 4) * 16
        tgt_vec = tgt_v[pl.ds(w0, _L)]
        lane_base = (wid % 4) * 4

        out_acc = jnp.zeros((_L,), jnp.float32)
        for j in range(_ROWS_PER_W):
            r = wid * _ROWS_PER_W + j
            pltpu.sync_copy(inp_hbm.at[r], row_v)

            lane = lane_base + j
            t_f = _xlane_sum(jnp.where(iota == lane, tgt_vec, 0)
                             .astype(jnp.float32))
            t_b = t_f.astype(jnp.int32)
            x_t = plsc.load_gather(row_v, [t_b])

            cmax_vec = neg_inf
            for c in range(_NCHUNK):
                lo = c * _CSIZE
                n = _CSIZE if c < _NCHUNK - 1 else (_NVEC - lo)

                def pass_a(i, pm, lo=lo):
                    vs = [row_v[pl.ds((lo + i * _U + u) * _L, _L)]
                          for u in range(_U)]
                    return jnp.maximum(pm, _tree_max(vs))

                pm = lax.fori_loop(0, n // _U, pass_a, neg_inf)
                cmax_vec = jnp.where(iota == c, _xlane_max(pm), cmax_vec)

            m_b = _xlane_max(cmax_vec)

            def pass_b(i, s):
                es = [jnp.exp(row_v[pl.ds((i * _U + u) * _L, _L)] - m_b)
                      for u in range(_U)]
                return s + _tree_sum(es)

            s16 = lax.fori_loop(0, _NVEC // _U, pass_b,
                                jnp.zeros((_L,), jnp.float32))
            s_b = _xlane_sum(s16)

            sk, sc_ids = plsc.sort_key_val(cmax_vec, iota, descending=True)

            t1 = t2_ = t3 = t4 = t5 = neg_inf
            for k in range(_TOPK):
                cid = sc_ids[k]
                lo = cid * _CSIZE
                n = jnp.where(cid == _NCHUNK - 1, _NVEC - (_NCHUNK - 1) * _CSIZE,
                              _CSIZE)

                def bubble(i, carry):
                    b1, b2, b3, b4, b5 = carry
                    v = row_v[pl.ds(i * _L, _L)]
                    m1 = jnp.maximum(b1, v)
                    r1 = jnp.minimum(b1, v)
                    m2 = jnp.maximum(b2, r1)
                    r2 = jnp.minimum(b2, r1)
                    m3 = jnp.maximum(b3, r2)
                    r3 = jnp.minimum(b3, r2)
                    m4 = jnp.maximum(b4, r3)
                    r4 = jnp.minimum(b4, r3)
                    m5 = jnp.maximum(b5, r4)
                    return (m1, m2, m3, m4, m5)

                t1, t2_, t3, t4, t5 = lax.fori_loop(
                    lo, lo + n, bubble, (t1, t2_, t3, t4, t5))

            vv = neg_inf
            for k in range(_TOPK):
                topv_b = _xlane_max(t1)
                vv = jnp.where(iota == k, topv_b, vv)
                if k < _TOPK - 1:
                    hit = t1 == topv_b
                    first = iota == plsc.all_reduce_ffs(hit)
                    t1 = jnp.where(first, t2_, t1)
                    t2_ = jnp.where(first, t3, t2_)
                    t3 = jnp.where(first, t4, t3)
                    t4 = jnp.where(first, t5, t4)
                    t5 = jnp.where(first, neg_inf, t5)

            pw = jnp.exp(vv * _LN_POWER)
            s5_b = _xlane_sum(pw)

            v5_s = vv[_TOPK - 1]
            x_t_s = x_t[0]

            def rank_pass():
                def cntf(i, cnt):
                    v = row_v[pl.ds(i * _L, _L)]
                    idx = iota + i * _L
                    pred = (v > x_t) | ((v == x_t) & (idx < t_b))
                    return cnt + pred.astype(jnp.float32)

                cnt = lax.fori_loop(0, _NVEC, cntf,
                                    jnp.zeros((_L,), jnp.float32), unroll=2)
                return _xlane_sum(cnt) < float(_TOPK)

            in_topk = lax.cond(x_t_s == v5_s, rank_pass,
                               lambda: jnp.full((_L,), x_t_s > v5_s))

            lse = m_b + _vlog(s_b)
            val2 = jnp.where(in_topk, x_t * _LN_POWER - _vlog(s5_b),
                             x_t - lse)
            contrib = -(x_t - lse) - val2
            out_acc = jnp.where(iota == j, contrib, out_acc)

        out_v[...] = out_acc
        pltpu.sync_copy(out_v, out_hbm.at[wid])

    return run(inp, target)


def kernel(input, target):
    per_worker = _sc_loss_call(input, target)
    per_row = per_worker[:, :_ROWS_PER_W].reshape(_B)
    return jnp.mean(per_row)


# speedup vs baseline: 2.0890x; 1.0062x over previous
import functools
import math

import jax
import jax.numpy as jnp
from jax import lax
from jax.experimental import pallas as pl
from jax.experimental.pallas import tpu as pltpu
from jax.experimental.pallas import tpu_sc as plsc

_TOPK = 5
_LN_POWER = math.log(1.01)
_L = 16
_B = 128
_V = 100000
_NVEC = _V // _L
_NW = 32
_ROWS_PER_W = _B // _NW
_LN2 = 0.6931471805599453
_U = 10
_NCHUNK = 16
_CSIZE = 390


def _take(x, idx):
    return x.at[idx].get(mode="promise_in_bounds")


def _xlane_sum(x):
    iota = lax.iota(jnp.int32, _L)
    for k in (8, 4, 2, 1):
        x = x + _take(x, iota ^ k)
    return x


def _xlane_max(x):
    iota = lax.iota(jnp.int32, _L)
    for k in (8, 4, 2, 1):
        x = jnp.maximum(x, _take(x, iota ^ k))
    return x


def _vlog(x):
    bits = plsc.bitcast(x, jnp.int32)
    e = lax.shift_right_logical(bits, 23) - 127
    mbits = (bits & 0x7FFFFF) | (127 << 23)
    m = plsc.bitcast(mbits, jnp.float32)
    big = m > 1.4142135623730951
    m = jnp.where(big, m * 0.5, m)
    e = e + big.astype(jnp.int32)
    t = (m - 1.0) / (m + 1.0)
    t2 = t * t
    p = 2.0 * t * (1.0 + t2 * (1.0 / 3.0 + t2 * (0.2 + t2 * (1.0 / 7.0))))
    return e.astype(jnp.float32) * _LN2 + p


def _tree_max(vs):
    while len(vs) > 1:
        vs = [jnp.maximum(vs[i], vs[i + 1]) for i in range(0, len(vs) - 1, 2)] \
            + ([vs[-1]] if len(vs) % 2 else [])
    return vs[0]


def _tree_sum(vs):
    while len(vs) > 1:
        vs = [vs[i] + vs[i + 1] for i in range(0, len(vs) - 1, 2)] \
            + ([vs[-1]] if len(vs) % 2 else [])
    return vs[0]


def _sc_loss_call(inp, target):
    mesh = plsc.VectorSubcoreMesh(core_axis_name="c", subcore_axis_name="s")

    @functools.partial(
        pl.kernel,
        out_type=jax.ShapeDtypeStruct((_NW, _L), jnp.float32),
        mesh=mesh,
        compiler_params=pltpu.CompilerParams(needs_layout_passes=False,
                                             use_tc_tiling_on_sc=True),
        scratch_types=[
            pltpu.VMEM((_V,), jnp.float32),
            pltpu.VMEM((_B,), jnp.int32),
            pltpu.VMEM((_L,), jnp.float32),
        ],
    )
    def run(inp_hbm, tgt_hbm, out_hbm, row_v, tgt_v, out_v):
        wid = lax.axis_index("s") * 2 + lax.axis_index("c")
        iota = lax.iota(jnp.int32, _L)
        neg_inf = jnp.full((_L,), -jnp.inf, jnp.float32)

        pltpu.sync_copy(tgt_hbm, tgt_v)
        w0 = (wid // 4) * 16
        tgt_vec = tgt_v[pl.ds(w0, _L)]
        lane_base = (wid % 4) * 4

        out_acc = jnp.zeros((_L,), jnp.float32)
        for j in range(_ROWS_PER_W):
            r = wid * _ROWS_PER_W + j
            pltpu.sync_copy(inp_hbm.at[r], row_v)

            lane = lane_base + j
            t_f = _xlane_sum(jnp.where(iota == lane, tgt_vec, 0)
                             .astype(jnp.float32))
            t_b = t_f.astype(jnp.int32)
            x_t = plsc.load_gather(row_v, [t_b])

            cmax_vec = neg_inf
            for c in range(_NCHUNK):
                lo = c * _CSIZE
                n = _CSIZE if c < _NCHUNK - 1 else (_NVEC - lo)

                def pass_a(i, pm, lo=lo):
                    vs = [row_v[pl.ds((lo + i * _U + u) * _L, _L)]
                          for u in range(_U)]
                    return jnp.maximum(pm, _tree_max(vs))

                pm = lax.fori_loop(0, n // _U, pass_a, neg_inf)
                cmax_vec = jnp.where(iota == c, _xlane_max(pm), cmax_vec)

            m_b = _xlane_max(cmax_vec)

            def pass_b(i, s):
                es = [jnp.exp(row_v[pl.ds((i * _U + u) * _L, _L)] - m_b)
                      for u in range(_U)]
                return s + _tree_sum(es)

            s16 = lax.fori_loop(0, _NVEC // _U, pass_b,
                                jnp.zeros((_L,), jnp.float32))
            s_b = _xlane_sum(s16)

            sk, sc_ids = plsc.sort_key_val(cmax_vec, iota, descending=True)

            t1 = t2_ = t3 = t4 = t5 = neg_inf
            for k in range(_TOPK):
                cid = sc_ids[k]
                lo = cid * _CSIZE
                n = jnp.where(cid == _NCHUNK - 1, _NVEC - (_NCHUNK - 1) * _CSIZE,
                              _CSIZE)

                def bubble(i, carry):
                    b1, b2, b3, b4, b5 = carry
                    v = row_v[pl.ds(i * _L, _L)]
                    m1 = jnp.maximum(b1, v)
                    r1 = jnp.minimum(b1, v)
                    m2 = jnp.maximum(b2, r1)
                    r2 = jnp.minimum(b2, r1)
                    m3 = jnp.maximum(b3, r2)
                    r3 = jnp.minimum(b3, r2)
                    m4 = jnp.maximum(b4, r3)
                    r4 = jnp.minimum(b4, r3)
                    m5 = jnp.maximum(b5, r4)
                    return (m1, m2, m3, m4, m5)

                t1, t2_, t3, t4, t5 = lax.fori_loop(
                    lo, lo + n, bubble, (t1, t2_, t3, t4, t5))

            vv = neg_inf
            for k in range(_TOPK):
                topv_b = _xlane_max(t1)
                vv = jnp.where(iota == k, topv_b, vv)
                if k < _TOPK - 1:
                    hit = t1 == topv_b
                    first = iota == plsc.all_reduce_ffs(hit)
                    t1 = jnp.where(first, t2_, t1)
                    t2_ = jnp.where(first, t3, t2_)
                    t3 = jnp.where(first, t4, t3)
                    t4 = jnp.where(first, t5, t4)
                    t5 = jnp.where(first, neg_inf, t5)

            pw = jnp.exp(vv * _LN_POWER)
            s5_b = _xlane_sum(pw)

            v5_s = vv[_TOPK - 1]
            x_t_s = x_t[0]

            def rank_pass():
                def cntf(i, cnt):
                    v = row_v[pl.ds(i * _L, _L)]
                    idx = iota + i * _L
                    pred = (v > x_t) | ((v == x_t) & (idx < t_b))
                    return cnt + pred.astype(jnp.float32)

                cnt = lax.fori_loop(0, _NVEC, cntf,
                                    jnp.zeros((_L,), jnp.float32), unroll=2)
                return _xlane_sum(cnt) < float(_TOPK)

            in_topk = lax.cond(x_t_s == v5_s, rank_pass,
                               lambda: jnp.full((_L,), x_t_s > v5_s))

            lse = m_b + _vlog(s_b)
            val2 = jnp.where(in_topk, x_t * _LN_POWER - _vlog(s5_b),
                             x_t - lse)
            contrib = -(x_t - lse) - val2
            out_acc = jnp.where(iota == j, contrib, out_acc)

        out_v[...] = out_acc
        pltpu.sync_copy(out_v, out_hbm.at[wid])

    return run(inp, target)


def kernel(input, target):
    per_worker = _sc_loss_call(input, target)
    per_row = per_worker[:, :_ROWS_PER_W].reshape(_B)
    return jnp.mean(per_row)
